# baseline (device time: 392379 ns/iter reference)
import functools

import jax
from jax import lax
from jax.experimental import pallas as pl
from jax.experimental.pallas import tpu as pltpu

_N_CHUNKS = 16
_SLOTS = 4


def kernel(x, pi):
    _, n_rows, n_cols = x.shape
    rows_per = n_rows // _N_CHUNKS

    def body(
        x_ref,
        pi_ref,
        out_ref,
        send_buf,
        recv_buf,
        send_sems,
        recv_sems,
        fill_sem,
        drain_sem,
        copy_sem,
        credit_sem,
    ):
        my_x = lax.axis_index("x")
        my_y = lax.axis_index("y")
        my_z = lax.axis_index("z")
        other_z = 1 - my_z
        target_z = pi_ref[my_z]

        barrier_sem = pltpu.get_barrier_semaphore()
        pl.semaphore_signal(
            barrier_sem,
            inc=1,
            device_id=(my_x, my_y, other_z),
            device_id_type=pl.DeviceIdType.MESH,
        )
        pl.semaphore_wait(barrier_sem, 1)

        @pl.when(target_z != my_z)
        def _():
            def chunk(ref, c):
                return ref.at[:, pl.ds(c * rows_per, rows_per), :]

            def make_rdma(c):
                s = c % _SLOTS
                return pltpu.make_async_remote_copy(
                    src_ref=send_buf.at[s],
                    dst_ref=recv_buf.at[s],
                    send_sem=send_sems.at[s],
                    recv_sem=recv_sems.at[s],
                    device_id=(my_x, my_y, target_z),
                    device_id_type=pl.DeviceIdType.MESH,
                )

            rdmas = [make_rdma(c) for c in range(_N_CHUNKS)]

            for c in range(_SLOTS):
                fill = pltpu.make_async_copy(
                    chunk(x_ref, c), send_buf.at[c], fill_sem
                )
                fill.start()
                fill.wait()
                rdmas[c].start()

            for c in range(_N_CHUNKS):
                s = c % _SLOTS
                rdmas[c].wait_recv()
                drain = pltpu.make_async_copy(
                    recv_buf.at[s], chunk(out_ref, c), drain_sem
                )
                drain.start()
                drain.wait()
                pl.semaphore_signal(
                    credit_sem,
                    inc=1,
                    device_id=(my_x, my_y, target_z),
                    device_id_type=pl.DeviceIdType.MESH,
                )
                nxt = c + _SLOTS
                if nxt < _N_CHUNKS:
                    rdmas[c].wait_send()
                    fill = pltpu.make_async_copy(
                        chunk(x_ref, nxt), send_buf.at[s], fill_sem
                    )
                    fill.start()
                    fill.wait()
                    pl.semaphore_wait(credit_sem, 1)
                    rdmas[nxt].start()
            for c in range(_N_CHUNKS - _SLOTS, _N_CHUNKS):
                rdmas[c].wait_send()
            pl.semaphore_wait(credit_sem, _SLOTS)

        @pl.when(target_z == my_z)
        def _():
            copy = pltpu.make_async_copy(x_ref, out_ref, copy_sem)
            copy.start()
            copy.wait()

        @functools.partial(pl.run_scoped, exit_sem=pltpu.SemaphoreType.REGULAR)
        def _(exit_sem):
            pl.semaphore_signal(
                exit_sem,
                inc=1,
                device_id=(my_x, my_y, other_z),
                device_id_type=pl.DeviceIdType.MESH,
            )
            pl.semaphore_wait(exit_sem, 1)

    return pl.pallas_call(
        body,
        out_shape=jax.ShapeDtypeStruct(x.shape, x.dtype),
        in_specs=[
            pl.BlockSpec(memory_space=pl.ANY),
            pl.BlockSpec(memory_space=pltpu.SMEM),
        ],
        out_specs=pl.BlockSpec(memory_space=pl.ANY),
        scratch_shapes=[
            pltpu.VMEM((_SLOTS, 1, rows_per, n_cols), x.dtype),
            pltpu.VMEM((_SLOTS, 1, rows_per, n_cols), x.dtype),
            pltpu.SemaphoreType.DMA((_SLOTS,)),
            pltpu.SemaphoreType.DMA((_SLOTS,)),
            pltpu.SemaphoreType.DMA,
            pltpu.SemaphoreType.DMA,
            pltpu.SemaphoreType.DMA,
            pltpu.SemaphoreType.REGULAR,
        ],
        compiler_params=pltpu.CompilerParams(collective_id=0),
    )(x, pi)


# device time: 389443 ns/iter; 1.0075x vs baseline; 1.0075x over previous
import functools

import jax
from jax import lax
from jax.experimental import pallas as pl
from jax.experimental.pallas import tpu as pltpu


def kernel(x, pi):
    def body(x_ref, pi_ref, out_ref, send_sem, recv_sem, copy_sem):
        my_x = lax.axis_index("x")
        my_y = lax.axis_index("y")
        my_z = lax.axis_index("z")
        other_z = 1 - my_z
        target_z = pi_ref[my_z]

        barrier_sem = pltpu.get_barrier_semaphore()
        pl.semaphore_signal(
            barrier_sem,
            inc=1,
            device_id=(my_x, my_y, other_z),
            device_id_type=pl.DeviceIdType.MESH,
        )
        pl.semaphore_wait(barrier_sem, 1)

        @pl.when(target_z != my_z)
        def _():
            rdma = pltpu.make_async_remote_copy(
                src_ref=x_ref,
                dst_ref=out_ref,
                send_sem=send_sem,
                recv_sem=recv_sem,
                device_id=(my_x, my_y, target_z),
                device_id_type=pl.DeviceIdType.MESH,
            )
            rdma.start()
            rdma.wait()

        @pl.when(target_z == my_z)
        def _():
            copy = pltpu.make_async_copy(x_ref, out_ref, copy_sem)
            copy.start()
            copy.wait()

        @functools.partial(pl.run_scoped, exit_sem=pltpu.SemaphoreType.REGULAR)
        def _(exit_sem):
            pl.semaphore_signal(
                exit_sem,
                inc=1,
                device_id=(my_x, my_y, other_z),
                device_id_type=pl.DeviceIdType.MESH,
            )
            pl.semaphore_wait(exit_sem, 1)

    return pl.pallas_call(
        body,
        out_shape=jax.ShapeDtypeStruct(x.shape, x.dtype),
        in_specs=[
            pl.BlockSpec(memory_space=pl.ANY),
            pl.BlockSpec(memory_space=pltpu.SMEM),
        ],
        out_specs=pl.BlockSpec(memory_space=pl.ANY),
        scratch_shapes=[
            pltpu.SemaphoreType.DMA,
            pltpu.SemaphoreType.DMA,
            pltpu.SemaphoreType.DMA,
        ],
        compiler_params=pltpu.CompilerParams(collective_id=0),
    )(x, pi)
